# baseline (speedup 1.0000x reference)
import functools

import jax
import jax.numpy as jnp
from jax import lax
from jax.experimental import pallas as pl
from jax.experimental.pallas import tpu as pltpu
from jax.experimental.pallas import tpu_sc as plsc

N_CORES = 2
N_SUBCORES = 16
N_TILES = N_CORES * N_SUBCORES
LANES = 16
ROWW = 8
CHUNK = 800
NBUF = 2


def _sc_accumulate(charges_flat, nbr_flat, dists, na):
    npairs = dists.shape[0]
    ppt = npairs // N_TILES
    nchunks = ppt // CHUNK
    trows = na // N_SUBCORES

    mesh = plsc.VectorSubcoreMesh(core_axis_name="c", subcore_axis_name="s")

    @functools.partial(
        pl.kernel,
        mesh=mesh,
        compiler_params=pltpu.CompilerParams(
            needs_layout_passes=False, use_tc_tiling_on_sc=False),
        out_type=jax.ShapeDtypeStruct((N_CORES * na, ROWW), jnp.float32),
        scratch_types=[
            pltpu.VMEM_SHARED((na, ROWW), jnp.float32),
            pltpu.VMEM_SHARED((na, ROWW), jnp.float32),
            pltpu.VMEM((CHUNK,), jnp.int32),
            pltpu.VMEM((NBUF, CHUNK), jnp.int32),
            pltpu.VMEM((NBUF, CHUNK), jnp.int32),
            pltpu.VMEM((CHUNK,), jnp.float32),
            pltpu.VMEM((NBUF, CHUNK, ROWW), jnp.float32),
            pltpu.VMEM((NBUF, CHUNK, ROWW), jnp.float32),
            pltpu.SemaphoreType.DMA,
            pltpu.SemaphoreType.DMA,
            pltpu.SemaphoreType.DMA,
            pltpu.SemaphoreType.DMA,
        ],
    )
    def run(cf_hbm, nbr_hbm, dd_hbm, out_hbm,
            ch_sp, acc_sp, nbr_v, ii_v, jj_v, dd_v, val_i, val_j,
            semL, semG, semS0, semS1):
        c = lax.axis_index("c")
        s = lax.axis_index("s")
        wid = s * N_CORES + c
        lane = lax.iota(jnp.int32, LANES)
        l4 = lane >> 2
        lm = lane & 3
        l8 = lane >> 3
        lw = lane & 7
        zeros = jnp.zeros((LANES,), jnp.float32)
        semS = (semS0, semS1)

        zbuf = val_j.at[0]
        sbuf = val_i.at[0]

        def zero2(k, carry):
            plsc.store_scatter(zbuf, [2 * k + l8, lw], zeros)
            plsc.store_scatter(sbuf, [2 * k + l8, lw], zeros)
            return carry
        lax.fori_loop(0, CHUNK * ROWW // LANES, zero2, 0)

        row0 = s * trows
        SROWS = CHUNK // 4

        def over_slices(total, step, fn):
            off = 0
            while off < total:
                n = min(step, total - off)
                fn(off, n)
                off += n

        def init(off, n):
            pltpu.sync_copy(cf_hbm.at[pl.ds(4 * (row0 + off), 4 * n)],
                            dd_v.at[pl.ds(0, 4 * n)])

            def spread(k, carry):
                w = 16 * k + lane
                m = w < 4 * n
                v = plsc.load_gather(dd_v, [w], mask=m)
                plsc.store_scatter(sbuf, [w >> 2, lm], v, mask=m)
                return carry
            lax.fori_loop(0, (4 * n + LANES - 1) // LANES, spread, 0)
            pltpu.sync_copy(zbuf.at[pl.ds(0, n)],
                            acc_sp.at[pl.ds(row0 + off, n)])
            pltpu.sync_copy(sbuf.at[pl.ds(0, n)],
                            ch_sp.at[pl.ds(row0 + off, n)])
        over_slices(trows, SROWS, init)

        plsc.subcore_barrier()

        def scat_desc(b):
            return (pltpu.make_async_copy(val_j.at[b], acc_sp.at[ii_v.at[b]],
                                          semS[b]),
                    pltpu.make_async_copy(val_i.at[b], acc_sp.at[jj_v.at[b]],
                                          semS[b]))

        def process(t, b):
            base = wid * ppt + t * CHUNK
            ld1 = pltpu.async_copy(nbr_hbm.at[pl.ds(2 * base, CHUNK)],
                                   nbr_v, semL)
            @pl.when(t >= NBUF)
            def _():
                d1, d2 = scat_desc(b)
                d1.wait()
                d2.wait()

            iv = ii_v.at[b]
            jv = jj_v.at[b]

            def deint(half, ld):
                ld.wait()

                def dbody(k, cr):
                    w = 32 * k + 2 * lane
                    i16 = plsc.load_gather(nbr_v, [w])
                    j16 = plsc.load_gather(nbr_v, [w + 1])
                    sl = pl.ds(half * (CHUNK // 2) + k * LANES, LANES)
                    iv[sl] = i16
                    jv[sl] = j16
                    return cr
                lax.fori_loop(0, CHUNK // 2 // LANES, dbody, 0)

            deint(0, ld1)
            nxt = pltpu.async_copy(
                nbr_hbm.at[pl.ds(2 * base + CHUNK, CHUNK)], nbr_v, semL)
            deint(1, nxt)

            g1 = pltpu.async_copy(ch_sp.at[jv], val_j.at[b], semG)
            g2 = pltpu.async_copy(ch_sp.at[iv], val_i.at[b], semG)
            ld2 = pltpu.async_copy(dd_hbm.at[pl.ds(base, CHUNK)],
                                   dd_v, semL)
            g1.wait()
            g2.wait()
            ld2.wait()

            vj = val_j.at[b]
            vi = val_i.at[b]

            def mbody(k, cr):
                pidx = 4 * k + l4
                p = 0.5 / plsc.load_gather(dd_v, [pidx])
                rj = plsc.load_gather(vj, [pidx, lm])
                ri = plsc.load_gather(vi, [pidx, lm])
                plsc.store_scatter(vj, [pidx, lm], rj * p)
                plsc.store_scatter(vi, [pidx, lm], ri * p)
                return cr
            lax.fori_loop(0, CHUNK * 4 // LANES, mbody, 0)

            d1, d2 = scat_desc(b)
            d1.start(add=True)
            d2.start(add=True)

        def outer(g, carry):
            for b in range(NBUF):
                process(g * NBUF + b, b)
            return carry
        lax.fori_loop(0, nchunks // NBUF, outer, 0)

        for b in range(NBUF):
            d1, d2 = scat_desc(b)
            d1.wait()
            d2.wait()

        plsc.subcore_barrier()

        def write(off, n):
            pltpu.sync_copy(acc_sp.at[pl.ds(row0 + off, n)],
                            sbuf.at[pl.ds(0, n)])
            pltpu.sync_copy(sbuf.at[pl.ds(0, n)],
                            out_hbm.at[pl.ds(c * na + row0 + off, n)])
        over_slices(trows, CHUNK, write)

    return run(charges_flat, nbr_flat, dists)


def _tc_add_halves(parts2d, rows):
    def body(a_ref, o_ref):
        o_ref[...] = a_ref[pl.ds(0, rows), :] + a_ref[pl.ds(rows, rows), :]
    return pl.pallas_call(
        body, out_shape=jax.ShapeDtypeStruct((rows, 128), parts2d.dtype),
    )(parts2d)


def kernel(charges, cell, positions, neighbor_indices, neighbor_distances):
    na, ch = charges.shape
    parts = _sc_accumulate(charges.reshape(na * ch),
                           neighbor_indices.reshape(-1),
                           neighbor_distances, na)
    rows = na * ROWW // 128
    summed = _tc_add_halves(parts.reshape(2 * rows, 128), rows)
    return summed.reshape(na, ROWW)[:, :ch]


# speedup vs baseline: 7.9425x; 7.9425x over previous
import functools

import jax
import jax.numpy as jnp
from jax import lax
from jax.experimental import pallas as pl
from jax.experimental.pallas import tpu as pltpu
from jax.experimental.pallas import tpu_sc as plsc

N_CORES = 2
N_SUBCORES = 16
N_TILES = N_CORES * N_SUBCORES
LANES = 16
ROWW = 8
CHUNK = 800
NBUF = 2


def _sc_accumulate(charges8, idx_i, idx_j, dists):
    na = charges8.shape[0]
    npairs = dists.shape[0]
    ppt = npairs // N_TILES
    nchunks = ppt // CHUNK
    trows = na // N_SUBCORES

    mesh = plsc.VectorSubcoreMesh(core_axis_name="c", subcore_axis_name="s")

    @functools.partial(
        pl.kernel,
        mesh=mesh,
        compiler_params=pltpu.CompilerParams(
            needs_layout_passes=False, use_tc_tiling_on_sc=False),
        out_type=jax.ShapeDtypeStruct((N_CORES * na, ROWW), jnp.float32),
        scratch_types=[
            pltpu.VMEM_SHARED((na, ROWW), jnp.float32),
            pltpu.VMEM_SHARED((na, ROWW), jnp.float32),
            pltpu.VMEM((NBUF, CHUNK), jnp.int32),
            pltpu.VMEM((NBUF, CHUNK), jnp.int32),
            pltpu.VMEM((CHUNK,), jnp.float32),
            pltpu.VMEM((NBUF, CHUNK, ROWW), jnp.float32),
            pltpu.VMEM((NBUF, CHUNK, ROWW), jnp.float32),
            pltpu.SemaphoreType.DMA,
            pltpu.SemaphoreType.DMA,
            pltpu.SemaphoreType.DMA,
            pltpu.SemaphoreType.DMA,
        ],
    )
    def run(ch_hbm, ii_hbm, jj_hbm, dd_hbm, out_hbm,
            ch_sp, acc_sp, ii_v, jj_v, dd_v, val_i, val_j,
            semL, semG, semS0, semS1):
        c = lax.axis_index("c")
        s = lax.axis_index("s")
        wid = s * N_CORES + c
        lane = lax.iota(jnp.int32, LANES)
        l4 = lane >> 2
        lm = lane & 3
        l8 = lane >> 3
        lw = lane & 7
        zeros = jnp.zeros((LANES,), jnp.float32)
        semS = (semS0, semS1)

        zbuf = val_j.at[0]
        sbuf = val_i.at[0]

        def zbody(k, carry):
            plsc.store_scatter(zbuf, [2 * k + l8, lw], zeros)
            return carry
        lax.fori_loop(0, CHUNK * ROWW // LANES, zbody, 0)

        row0 = s * trows

        def over_slices(total, fn):
            off = 0
            while off < total:
                n = min(CHUNK, total - off)
                fn(off, n)
                off += n

        def init(off, n):
            pltpu.sync_copy(zbuf.at[pl.ds(0, n)],
                            acc_sp.at[pl.ds(row0 + off, n)])
            pltpu.sync_copy(ch_hbm.at[pl.ds(row0 + off, n)],
                            sbuf.at[pl.ds(0, n)])
            pltpu.sync_copy(sbuf.at[pl.ds(0, n)],
                            ch_sp.at[pl.ds(row0 + off, n)])
        over_slices(trows, init)

        plsc.subcore_barrier()

        def scat_desc(b):
            return (pltpu.make_async_copy(val_j.at[b], acc_sp.at[ii_v.at[b]],
                                          semS[b]),
                    pltpu.make_async_copy(val_i.at[b], acc_sp.at[jj_v.at[b]],
                                          semS[b]))

        def process(t, b):
            base = wid * ppt + t * CHUNK
            ld1 = pltpu.async_copy(ii_hbm.at[pl.ds(base, CHUNK)],
                                   ii_v.at[b], semL)
            ld2 = pltpu.async_copy(jj_hbm.at[pl.ds(base, CHUNK)],
                                   jj_v.at[b], semL)
            @pl.when(t >= NBUF)
            def _():
                d1, d2 = scat_desc(b)
                d1.wait()
                d2.wait()
            ld1.wait()
            ld2.wait()
            g1 = pltpu.async_copy(ch_sp.at[jj_v.at[b]], val_j.at[b], semG)
            g2 = pltpu.async_copy(ch_sp.at[ii_v.at[b]], val_i.at[b], semG)
            ld3 = pltpu.async_copy(dd_hbm.at[pl.ds(base, CHUNK)], dd_v, semL)
            g1.wait()
            g2.wait()
            ld3.wait()

            vj = val_j.at[b]
            vi = val_i.at[b]

            def mbody(k, cr):
                pidx = 4 * k + l4
                p = 0.5 / plsc.load_gather(dd_v, [pidx])
                rj = plsc.load_gather(vj, [pidx, lm])
                ri = plsc.load_gather(vi, [pidx, lm])
                plsc.store_scatter(vj, [pidx, lm], rj * p)
                plsc.store_scatter(vi, [pidx, lm], ri * p)
                return cr
            lax.fori_loop(0, CHUNK * 4 // LANES, mbody, 0)

            d1, d2 = scat_desc(b)
            d1.start(add=True)
            d2.start(add=True)

        def outer(g, carry):
            for b in range(NBUF):
                process(g * NBUF + b, b)
            return carry
        lax.fori_loop(0, nchunks // NBUF, outer, 0)

        for b in range(NBUF):
            d1, d2 = scat_desc(b)
            d1.wait()
            d2.wait()

        plsc.subcore_barrier()

        def write(off, n):
            pltpu.sync_copy(acc_sp.at[pl.ds(row0 + off, n)],
                            sbuf.at[pl.ds(0, n)])
            pltpu.sync_copy(sbuf.at[pl.ds(0, n)],
                            out_hbm.at[pl.ds(c * na + row0 + off, n)])
        over_slices(trows, write)

    return run(charges8, idx_i, idx_j, dists)


def _tc_add_halves(parts2d, rows):
    def body(a_ref, o_ref):
        o_ref[...] = a_ref[pl.ds(0, rows), :] + a_ref[pl.ds(rows, rows), :]
    return pl.pallas_call(
        body, out_shape=jax.ShapeDtypeStruct((rows, 128), parts2d.dtype),
    )(parts2d)


def kernel(charges, cell, positions, neighbor_indices, neighbor_distances):
    na, ch = charges.shape
    idx_i = neighbor_indices[:, 0]
    idx_j = neighbor_indices[:, 1]
    charges8 = jnp.pad(charges, ((0, 0), (0, ROWW - ch)))
    parts = _sc_accumulate(charges8, idx_i, idx_j, neighbor_distances)
    rows = na * ROWW // 128
    summed = _tc_add_halves(parts.reshape(2 * rows, 128), rows)
    return summed.reshape(na, ROWW)[:, :ch]


# speedup vs baseline: 8.7815x; 1.1056x over previous
import functools

import jax
import jax.numpy as jnp
from jax import lax
from jax.experimental import pallas as pl
from jax.experimental.pallas import tpu as pltpu
from jax.experimental.pallas import tpu_sc as plsc

N_CORES = 2
N_SUBCORES = 16
N_TILES = N_CORES * N_SUBCORES
LANES = 16
ROWW = 8
CHUNK = 400
NBUF = 3


def _sc_accumulate(charges8, idx_i, idx_j, dists):
    na = charges8.shape[0]
    npairs = dists.shape[0]
    ppt = npairs // N_TILES
    nchunks = ppt // CHUNK
    trows = na // N_SUBCORES

    mesh = plsc.VectorSubcoreMesh(core_axis_name="c", subcore_axis_name="s")

    @functools.partial(
        pl.kernel,
        mesh=mesh,
        compiler_params=pltpu.CompilerParams(
            needs_layout_passes=False, use_tc_tiling_on_sc=False),
        out_type=jax.ShapeDtypeStruct((N_CORES * na, ROWW), jnp.float32),
        scratch_types=[
            pltpu.VMEM_SHARED((na, ROWW), jnp.float32),
            pltpu.VMEM_SHARED((na, ROWW), jnp.float32),
            pltpu.VMEM((NBUF, CHUNK), jnp.int32),
            pltpu.VMEM((NBUF, CHUNK), jnp.int32),
            pltpu.VMEM((NBUF, CHUNK), jnp.float32),
            pltpu.VMEM((NBUF, CHUNK, ROWW), jnp.float32),
            pltpu.VMEM((NBUF, CHUNK, ROWW), jnp.float32),
            pltpu.SemaphoreType.DMA,
            pltpu.SemaphoreType.DMA,
            pltpu.SemaphoreType.DMA,
            pltpu.SemaphoreType.DMA,
            pltpu.SemaphoreType.DMA,
            pltpu.SemaphoreType.DMA,
            pltpu.SemaphoreType.DMA,
        ],
    )
    def run(ch_hbm, ii_hbm, jj_hbm, dd_hbm, out_hbm,
            ch_sp, acc_sp, ii_v, jj_v, dd_v, val_i, val_j,
            semL0, semL1, semL2, semG, semS0, semS1, semS2):
        c = lax.axis_index("c")
        s = lax.axis_index("s")
        wid = s * N_CORES + c
        lane = lax.iota(jnp.int32, LANES)
        l4 = lane >> 2
        lm = lane & 3
        l8 = lane >> 3
        lw = lane & 7
        zeros = jnp.zeros((LANES,), jnp.float32)
        semL = (semL0, semL1, semL2)
        semS = (semS0, semS1, semS2)

        zbuf = val_j.at[0]
        sbuf = val_i.at[0]

        def zbody(k, carry):
            plsc.store_scatter(zbuf, [2 * k + l8, lw], zeros)
            return carry
        lax.fori_loop(0, CHUNK * ROWW // LANES, zbody, 0)

        row0 = s * trows

        def over_slices(total, fn):
            off = 0
            while off < total:
                n = min(CHUNK, total - off)
                fn(off, n)
                off += n

        def init(off, n):
            pltpu.sync_copy(zbuf.at[pl.ds(0, n)],
                            acc_sp.at[pl.ds(row0 + off, n)])
            pltpu.sync_copy(ch_hbm.at[pl.ds(row0 + off, n)],
                            sbuf.at[pl.ds(0, n)])
            pltpu.sync_copy(sbuf.at[pl.ds(0, n)],
                            ch_sp.at[pl.ds(row0 + off, n)])
        over_slices(trows, init)

        plsc.subcore_barrier()

        def scat_desc(b):
            return (pltpu.make_async_copy(val_j.at[b], acc_sp.at[ii_v.at[b]],
                                          semS[b]),
                    pltpu.make_async_copy(val_i.at[b], acc_sp.at[jj_v.at[b]],
                                          semS[b]))

        def load_descs(t, b):
            base = wid * ppt + t * CHUNK
            return (pltpu.make_async_copy(ii_hbm.at[pl.ds(base, CHUNK)],
                                          ii_v.at[b], semL[b]),
                    pltpu.make_async_copy(jj_hbm.at[pl.ds(base, CHUNK)],
                                          jj_v.at[b], semL[b]),
                    pltpu.make_async_copy(dd_hbm.at[pl.ds(base, CHUNK)],
                                          dd_v.at[b], semL[b]))

        def gath_descs(b):
            return (pltpu.make_async_copy(ch_sp.at[jj_v.at[b]], val_j.at[b],
                                          semG),
                    pltpu.make_async_copy(ch_sp.at[ii_v.at[b]], val_i.at[b],
                                          semG))

        def start_all(descs, **kw):
            for d in descs:
                d.start(**kw)

        def wait_all(descs):
            for d in descs:
                d.wait()

        lds = load_descs(0, 0)
        start_all(lds)
        start_all(load_descs(1, 1))
        wait_all(lds)
        start_all(gath_descs(0))

        def step(t, b):
            nxt = (b + 1) % NBUF
            n2 = (b + 2) % NBUF
            wait_all(gath_descs(b))

            vj = val_j.at[b]
            vi = val_i.at[b]
            dv = dd_v.at[b]

            def mbody(k, cr):
                pidx = 4 * k + l4
                p = 0.5 / plsc.load_gather(dv, [pidx])
                rj = plsc.load_gather(vj, [pidx, lm])
                ri = plsc.load_gather(vi, [pidx, lm])
                plsc.store_scatter(vj, [pidx, lm], rj * p)
                plsc.store_scatter(vi, [pidx, lm], ri * p)
                return cr
            lax.fori_loop(0, CHUNK * 4 // LANES, mbody, 0)

            start_all(scat_desc(b), add=True)

            @pl.when(t + 2 < nchunks)
            def _():
                @pl.when(t >= 1)
                def _():
                    wait_all(scat_desc(n2))
                start_all(load_descs(t + 2, n2))

            @pl.when(t + 1 < nchunks)
            def _():
                wait_all(load_descs(t + 1, nxt))
                start_all(gath_descs(nxt))

        def chunk_body(t, carry):
            bmod = lax.rem(t, NBUF)
            for bb in range(NBUF):
                @pl.when(bmod == bb)
                def _():
                    step(t, bb)
            return carry
        lax.fori_loop(0, nchunks, chunk_body, 0)

        for b in range(NBUF):
            wait_all(scat_desc(b))

        plsc.subcore_barrier()

        def write(off, n):
            pltpu.sync_copy(acc_sp.at[pl.ds(row0 + off, n)],
                            sbuf.at[pl.ds(0, n)])
            pltpu.sync_copy(sbuf.at[pl.ds(0, n)],
                            out_hbm.at[pl.ds(c * na + row0 + off, n)])
        over_slices(trows, write)

    return run(charges8, idx_i, idx_j, dists)


def _tc_add_halves(parts2d, rows):
    def body(a_ref, o_ref):
        o_ref[...] = a_ref[pl.ds(0, rows), :] + a_ref[pl.ds(rows, rows), :]
    return pl.pallas_call(
        body, out_shape=jax.ShapeDtypeStruct((rows, 128), parts2d.dtype),
    )(parts2d)


def kernel(charges, cell, positions, neighbor_indices, neighbor_distances):
    na, ch = charges.shape
    idx_i = neighbor_indices[:, 0]
    idx_j = neighbor_indices[:, 1]
    charges8 = jnp.pad(charges, ((0, 0), (0, ROWW - ch)))
    parts = _sc_accumulate(charges8, idx_i, idx_j, neighbor_distances)
    rows = na * ROWW // 128
    summed = _tc_add_halves(parts.reshape(2 * rows, 128), rows)
    return summed.reshape(na, ROWW)[:, :ch]
